# Initial kernel scaffold; baseline (speedup 1.0000x reference)
import functools

import jax
import jax.numpy as jnp
from jax import lax
from jax.experimental import pallas as pl
from jax.experimental.pallas import tpu as pltpu
from jax.experimental.pallas import tpu_sc as plsc

NC = 2
NS = 16
NW = NC * NS
CH = 128


def _fill_1d(ref, n, value):
  v = jnp.full((16,), value, dtype=jnp.float32)

  def body(i, _):
    ref[pl.ds(i * 16, 16)] = v
    return 0

  lax.fori_loop(0, n // 16, body, 0)


def _fill_2d(ref, rows, cols, value):
  v = jnp.full((16,), value, dtype=jnp.float32)

  def body(i, _):
    for j in range(cols // 16):
      ref[i, pl.ds(j * 16, 16)] = v
    return 0

  lax.fori_loop(0, rows, body, 0)


def _deg_body(ept, rpt, dst_hbm, deg_hbm, deg_acc, idx_v, ones_v, zeros_v):
  cid = lax.axis_index("c")
  sid = lax.axis_index("s")
  wid = cid * NS + sid

  _fill_1d(ones_v, CH, 1.0)
  _fill_1d(zeros_v, rpt, 0.0)
  pltpu.sync_copy(zeros_v, deg_acc.at[pl.ds(sid * rpt, rpt)])
  plsc.subcore_barrier()

  def step(k, _):
    base = wid * ept + k * CH
    pltpu.sync_copy(dst_hbm.at[pl.ds(base, CH)], idx_v)
    pltpu.sync_copy(ones_v, deg_acc.at[idx_v], add=True)
    return 0

  lax.fori_loop(0, ept // CH, step, 0)
  plsc.subcore_barrier()
  pltpu.sync_copy(deg_acc.at[pl.ds(sid * rpt, rpt)],
                  deg_hbm.at[cid, pl.ds(sid * rpt, rpt)])


def _agg_body(ept, rpt, src_hbm, dst_hbm, hs_hbm, part_hbm,
              acc, sidx_v, didx_v, rows_v, zrows_v, sem):
  cid = lax.axis_index("c")
  sid = lax.axis_index("s")
  wid = cid * NS + sid

  _fill_2d(zrows_v, CH, 128, 0.0)
  for r in range(rpt // CH):
    pltpu.sync_copy(zrows_v, acc.at[pl.ds(sid * rpt + r * CH, CH)])
  plsc.subcore_barrier()

  def step(k, _):
    base = wid * ept + k * CH
    pltpu.sync_copy(src_hbm.at[pl.ds(base, CH)], sidx_v)
    pltpu.sync_copy(dst_hbm.at[pl.ds(base, CH)], didx_v)
    pltpu.async_copy(hs_hbm.at[sidx_v], rows_v, sem).wait()
    pltpu.sync_copy(rows_v, acc.at[didx_v], add=True)
    return 0

  lax.fori_loop(0, ept // CH, step, 0)
  plsc.subcore_barrier()
  pltpu.sync_copy(acc.at[pl.ds(sid * rpt, rpt)],
                  part_hbm.at[cid, pl.ds(sid * rpt, rpt)])


def _hs_body(x_ref, w_ref, degp_ref, hs_ref):
  deg = degp_ref[0, :] + degp_ref[1, :] + 1.0
  dinv = lax.rsqrt(deg)
  h = jnp.dot(x_ref[...], w_ref[...], preferred_element_type=jnp.float32)
  hs_ref[...] = h * dinv[:, None]


def _out_body(part_ref, hs_ref, degp_ref, b_ref, out_ref):
  deg = degp_ref[0, :] + degp_ref[1, :] + 1.0
  dinv = lax.rsqrt(deg)
  s = part_ref[0] + part_ref[1] + hs_ref[...]
  out_ref[...] = s * dinv[:, None] + b_ref[0, :]


def kernel(x, edge_index, W, b):
  n, d = x.shape
  e = edge_index.shape[1]

  rpt = -(-(n + 1) // NS)
  rpt = -(-rpt // CH) * CH
  np_ = NS * rpt
  ept = -(-e // (NW * CH)) * CH
  ep = NW * ept

  src = jnp.concatenate(
      [edge_index[0], jnp.full((ep - e,), n, dtype=jnp.int32)])
  dst = jnp.concatenate(
      [edge_index[1], jnp.full((ep - e,), n, dtype=jnp.int32)])
  xp = jnp.concatenate(
      [x, jnp.zeros((np_ - n, d), dtype=x.dtype)])

  mesh = plsc.VectorSubcoreMesh(core_axis_name="c", subcore_axis_name="s")

  deg_kernel = pl.kernel(
      functools.partial(_deg_body, ept, rpt),
      out_type=jax.ShapeDtypeStruct((NC, np_), jnp.float32),
      mesh=mesh,
      scratch_types=[
          pltpu.VMEM_SHARED((np_,), jnp.float32),
          pltpu.VMEM((CH,), jnp.int32),
          pltpu.VMEM((CH,), jnp.float32),
          pltpu.VMEM((rpt,), jnp.float32),
      ],
  )
  degp = deg_kernel(dst)

  blk1 = 512
  hs = pl.pallas_call(
      _hs_body,
      grid=(np_ // blk1,),
      in_specs=[
          pl.BlockSpec((blk1, d), lambda i: (i, 0)),
          pl.BlockSpec((d, d), lambda i: (0, 0)),
          pl.BlockSpec((NC, blk1), lambda i: (0, i)),
      ],
      out_specs=pl.BlockSpec((blk1, d), lambda i: (i, 0)),
      out_shape=jax.ShapeDtypeStruct((np_, d), jnp.float32),
  )(xp, W, degp)

  agg_kernel = pl.kernel(
      functools.partial(_agg_body, ept, rpt),
      out_type=jax.ShapeDtypeStruct((NC, np_, d), jnp.float32),
      mesh=mesh,
      scratch_types=[
          pltpu.VMEM_SHARED((np_, d), jnp.float32),
          pltpu.VMEM((CH,), jnp.int32),
          pltpu.VMEM((CH,), jnp.int32),
          pltpu.VMEM((CH, d), jnp.float32),
          pltpu.VMEM((CH, d), jnp.float32),
          pltpu.SemaphoreType.DMA,
      ],
  )
  part = agg_kernel(src, dst, hs)

  blk2 = 2000
  out = pl.pallas_call(
      _out_body,
      grid=(n // blk2,),
      in_specs=[
          pl.BlockSpec((NC, blk2, d), lambda i: (0, i, 0)),
          pl.BlockSpec((blk2, d), lambda i: (i, 0)),
          pl.BlockSpec((NC, blk2), lambda i: (0, i)),
          pl.BlockSpec((1, d), lambda i: (0, 0)),
      ],
      out_specs=pl.BlockSpec((blk2, d), lambda i: (i, 0)),
      out_shape=jax.ShapeDtypeStruct((n, d), jnp.float32),
  )(part, hs, degp, b.reshape(1, d))

  return out


# speedup vs baseline: 16.6174x; 16.6174x over previous
import functools

import jax
import jax.numpy as jnp
from jax import lax
from jax.experimental import pallas as pl
from jax.experimental.pallas import tpu as pltpu
from jax.experimental.pallas import tpu_sc as plsc

NC = 2
NS = 16
NW = NC * NS
CH = 128


def _fill_1d(ref, n, value):
  v = jnp.full((16,), value, dtype=jnp.float32)

  def body(i, _):
    ref[pl.ds(i * 16, 16)] = v
    return 0

  lax.fori_loop(0, n // 16, body, 0)


def _fill_2d(ref, rows, cols, value):
  v = jnp.full((16,), value, dtype=jnp.float32)

  def body(i, _):
    for j in range(cols // 16):
      ref[i, pl.ds(j * 16, 16)] = v
    return 0

  lax.fori_loop(0, rows, body, 0)


def _deg_body(ept, rpt, dst_hbm, deg_hbm, deg_acc, idx_v, ones_v, zeros_v):
  cid = lax.axis_index("c")
  sid = lax.axis_index("s")
  wid = cid * NS + sid

  _fill_1d(ones_v, CH, 1.0)
  _fill_1d(zeros_v, rpt, 0.0)
  pltpu.sync_copy(zeros_v, deg_acc.at[pl.ds(sid * rpt, rpt)])
  plsc.subcore_barrier()

  def step(k, _):
    base = wid * ept + k * CH
    pltpu.sync_copy(dst_hbm.at[pl.ds(base, CH)], idx_v)
    pltpu.sync_copy(ones_v, deg_acc.at[idx_v], add=True)
    return 0

  lax.fori_loop(0, ept // CH, step, 0)
  plsc.subcore_barrier()
  pltpu.sync_copy(deg_acc.at[pl.ds(sid * rpt, rpt)],
                  deg_hbm.at[cid, pl.ds(sid * rpt, rpt)])


def _agg_body(ept, rpt, src_hbm, dst_hbm, hs_hbm, part_hbm,
              acc, sidx_v, didx_v, rows_v, zrows_v, sem):
  cid = lax.axis_index("c")
  sid = lax.axis_index("s")
  wid = cid * NS + sid

  _fill_2d(zrows_v, CH, 128, 0.0)
  for r in range(rpt // CH):
    pltpu.sync_copy(zrows_v, acc.at[pl.ds(sid * rpt + r * CH, CH)])
  plsc.subcore_barrier()

  def step(k, _):
    base = wid * ept + k * CH
    pltpu.sync_copy(src_hbm.at[pl.ds(base, CH)], sidx_v)
    pltpu.sync_copy(dst_hbm.at[pl.ds(base, CH)], didx_v)
    pltpu.async_copy(hs_hbm.at[sidx_v], rows_v, sem).wait()
    pltpu.sync_copy(rows_v, acc.at[didx_v], add=True)
    return 0

  lax.fori_loop(0, ept // CH, step, 0)
  plsc.subcore_barrier()
  pltpu.sync_copy(acc.at[pl.ds(sid * rpt, rpt)],
                  part_hbm.at[cid, pl.ds(sid * rpt, rpt)])


def _hs_body(x_ref, w_ref, degp_ref, hs_ref):
  deg = degp_ref[:, 0] + degp_ref[:, 1] + 1.0
  dinv = lax.rsqrt(deg)
  h = jnp.dot(x_ref[...], w_ref[...], preferred_element_type=jnp.float32)
  hs_ref[...] = h * dinv[:, None]


def _out_body(part_ref, hs_ref, degp_ref, b_ref, out_ref):
  deg = degp_ref[:, 0] + degp_ref[:, 1] + 1.0
  dinv = lax.rsqrt(deg)
  s = part_ref[0] + part_ref[1] + hs_ref[...]
  out_ref[...] = s * dinv[:, None] + b_ref[0, :]


def kernel(x, edge_index, W, b):
  n, d = x.shape
  e = edge_index.shape[1]

  rpt = -(-(n + 1) // NS)
  rpt = -(-rpt // CH) * CH
  np_ = NS * rpt
  ept = -(-e // (NW * CH)) * CH
  ep = NW * ept

  src = jnp.concatenate(
      [edge_index[0], jnp.full((ep - e,), n, dtype=jnp.int32)])
  dst = jnp.concatenate(
      [edge_index[1], jnp.full((ep - e,), n, dtype=jnp.int32)])
  xp = jnp.concatenate(
      [x, jnp.zeros((np_ - n, d), dtype=x.dtype)])

  mesh = plsc.VectorSubcoreMesh(core_axis_name="c", subcore_axis_name="s")

  deg_kernel = pl.kernel(
      functools.partial(_deg_body, ept, rpt),
      out_type=jax.ShapeDtypeStruct((NC, np_), jnp.float32),
      mesh=mesh,
      scratch_types=[
          pltpu.VMEM_SHARED((np_,), jnp.float32),
          pltpu.VMEM((CH,), jnp.int32),
          pltpu.VMEM((CH,), jnp.float32),
          pltpu.VMEM((rpt,), jnp.float32),
      ],
  )
  degp = deg_kernel(dst).T

  blk1 = 512
  hs = pl.pallas_call(
      _hs_body,
      grid=(np_ // blk1,),
      in_specs=[
          pl.BlockSpec((blk1, d), lambda i: (i, 0)),
          pl.BlockSpec((d, d), lambda i: (0, 0)),
          pl.BlockSpec((blk1, NC), lambda i: (i, 0)),
      ],
      out_specs=pl.BlockSpec((blk1, d), lambda i: (i, 0)),
      out_shape=jax.ShapeDtypeStruct((np_, d), jnp.float32),
  )(xp, W, degp)

  agg_kernel = pl.kernel(
      functools.partial(_agg_body, ept, rpt),
      out_type=jax.ShapeDtypeStruct((NC, np_, d), jnp.float32),
      mesh=mesh,
      scratch_types=[
          pltpu.VMEM_SHARED((np_, d), jnp.float32),
          pltpu.VMEM((CH,), jnp.int32),
          pltpu.VMEM((CH,), jnp.int32),
          pltpu.VMEM((CH, d), jnp.float32),
          pltpu.VMEM((CH, d), jnp.float32),
          pltpu.SemaphoreType.DMA,
      ],
  )
  part = agg_kernel(src, dst, hs)

  blk2 = 2000
  out = pl.pallas_call(
      _out_body,
      grid=(n // blk2,),
      in_specs=[
          pl.BlockSpec((NC, blk2, d), lambda i: (0, i, 0)),
          pl.BlockSpec((blk2, d), lambda i: (i, 0)),
          pl.BlockSpec((blk2, NC), lambda i: (i, 0)),
          pl.BlockSpec((1, d), lambda i: (0, 0)),
      ],
      out_specs=pl.BlockSpec((blk2, d), lambda i: (i, 0)),
      out_shape=jax.ShapeDtypeStruct((n, d), jnp.float32),
  )(part, hs, degp, b.reshape(1, d))

  return out


# speedup vs baseline: 17.6019x; 1.0592x over previous
import functools

import jax
import jax.numpy as jnp
from jax import lax
from jax.experimental import pallas as pl
from jax.experimental.pallas import tpu as pltpu
from jax.experimental.pallas import tpu_sc as plsc

NC = 2
NS = 16
NW = NC * NS
CH = 128


def _fill_1d(ref, n, value):
  v = jnp.full((16,), value, dtype=jnp.float32)

  def body(i, _):
    ref[pl.ds(i * 16, 16)] = v
    return 0

  lax.fori_loop(0, n // 16, body, 0)


def _fill_2d(ref, rows, cols, value):
  v = jnp.full((16,), value, dtype=jnp.float32)

  def body(i, _):
    for j in range(cols // 16):
      ref[i, pl.ds(j * 16, 16)] = v
    return 0

  lax.fori_loop(0, rows, body, 0)


def _deg_body(nk, rpt, dst_hbm, deg_hbm, deg_acc, didx_v, ones_v, zeros_v,
              sem):
  cid = lax.axis_index("c")
  sid = lax.axis_index("s")
  wid = cid * NS + sid

  _fill_1d(ones_v, CH, 1.0)
  _fill_1d(zeros_v, rpt, 0.0)
  pltpu.sync_copy(zeros_v, deg_acc.at[pl.ds(sid * rpt, rpt)])
  pltpu.sync_copy(dst_hbm.at[wid], didx_v)
  plsc.subcore_barrier()

  def fire(k, _):
    pltpu.async_copy(ones_v, deg_acc.at[didx_v.at[k]], sem, add=True)
    return 0

  lax.fori_loop(0, nk, fire, 0)

  def drain(k, _):
    pltpu.make_async_copy(ones_v, deg_acc.at[didx_v.at[0]], sem).wait()
    return 0

  lax.fori_loop(0, nk, drain, 0)
  plsc.subcore_barrier()
  pltpu.sync_copy(deg_acc.at[pl.ds(sid * rpt, rpt)],
                  deg_hbm.at[cid, pl.ds(sid * rpt, rpt)])


NB = 2
NI = 4


def _agg_body(nk, rpt, sd_hbm, hs_hbm, part_hbm,
              acc, idxb_v, rows_v, isem, gsem):
  cid = lax.axis_index("c")
  sid = lax.axis_index("s")
  wid = cid * NS + sid

  _fill_2d(rows_v.at[0], CH, 128, 0.0)
  for r in range(rpt // CH):
    pltpu.sync_copy(rows_v.at[0], acc.at[pl.ds(sid * rpt + r * CH, CH)])
  plsc.subcore_barrier()

  for i in range(NI):
    pltpu.async_copy(sd_hbm.at[wid, i], idxb_v.at[i], isem.at[i])
  for b in range(NB):
    pltpu.make_async_copy(
        sd_hbm.at[wid, b], idxb_v.at[b], isem.at[b]).wait()
    pltpu.async_copy(hs_hbm.at[idxb_v.at[b, 0]], rows_v.at[b], gsem.at[b])

  def group(g, _):
    for j in range(NI):
      k = g * NI + j
      b = j % NB
      i = j % NI
      pltpu.make_async_copy(
          hs_hbm.at[idxb_v.at[i, 0]], rows_v.at[b], gsem.at[b]).wait()
      pltpu.sync_copy(rows_v.at[b], acc.at[idxb_v.at[i, 1]], add=True)
      ki = k + NI

      @pl.when(ki < nk)
      def _():
        pltpu.async_copy(sd_hbm.at[wid, ki], idxb_v.at[i], isem.at[i])

      kg = k + NB
      ig = (k + NB) % NI

      @pl.when(kg < nk)
      def _():
        pltpu.make_async_copy(
            sd_hbm.at[wid, kg], idxb_v.at[ig], isem.at[ig]).wait()
        pltpu.async_copy(
            hs_hbm.at[idxb_v.at[ig, 0]], rows_v.at[b], gsem.at[b])

    return 0

  lax.fori_loop(0, nk // NI, group, 0)
  plsc.subcore_barrier()
  pltpu.sync_copy(acc.at[pl.ds(sid * rpt, rpt)],
                  part_hbm.at[cid, pl.ds(sid * rpt, rpt)])


def _hs_body(x_ref, w_ref, degp_ref, hs_ref):
  deg = degp_ref[:, 0] + degp_ref[:, 1] + 1.0
  dinv = lax.rsqrt(deg)
  h = jnp.dot(x_ref[...], w_ref[...], preferred_element_type=jnp.float32)
  hs_ref[...] = h * dinv[:, None]


def _out_body(part_ref, hs_ref, degp_ref, b_ref, out_ref):
  deg = degp_ref[:, 0] + degp_ref[:, 1] + 1.0
  dinv = lax.rsqrt(deg)
  s = part_ref[0] + part_ref[1] + hs_ref[...]
  out_ref[...] = s * dinv[:, None] + b_ref[0, :]


def kernel(x, edge_index, W, b):
  n, d = x.shape
  e = edge_index.shape[1]

  rpt = -(-(n + 1) // NS)
  rpt = -(-rpt // CH) * CH
  np_ = NS * rpt
  ept = -(-e // (NW * NI * CH)) * NI * CH
  ep = NW * ept
  nk = ept // CH

  src = jnp.concatenate(
      [edge_index[0], jnp.full((ep - e,), n, dtype=jnp.int32)]
  ).reshape(NW, nk, CH)
  dst = jnp.concatenate(
      [edge_index[1], jnp.full((ep - e,), n, dtype=jnp.int32)]
  ).reshape(NW, nk, CH)
  sd = jnp.stack([src, dst], axis=2)
  xp = jnp.concatenate(
      [x, jnp.zeros((np_ - n, d), dtype=x.dtype)])

  mesh = plsc.VectorSubcoreMesh(core_axis_name="c", subcore_axis_name="s")

  deg_kernel = pl.kernel(
      functools.partial(_deg_body, nk, rpt),
      out_type=jax.ShapeDtypeStruct((NC, np_), jnp.float32),
      mesh=mesh,
      scratch_types=[
          pltpu.VMEM_SHARED((np_,), jnp.float32),
          pltpu.VMEM((nk, CH), jnp.int32),
          pltpu.VMEM((CH,), jnp.float32),
          pltpu.VMEM((rpt,), jnp.float32),
          pltpu.SemaphoreType.DMA,
      ],
  )
  degp = deg_kernel(dst).T

  blk1 = 512
  hs = pl.pallas_call(
      _hs_body,
      grid=(np_ // blk1,),
      in_specs=[
          pl.BlockSpec((blk1, d), lambda i: (i, 0)),
          pl.BlockSpec((d, d), lambda i: (0, 0)),
          pl.BlockSpec((blk1, NC), lambda i: (i, 0)),
      ],
      out_specs=pl.BlockSpec((blk1, d), lambda i: (i, 0)),
      out_shape=jax.ShapeDtypeStruct((np_, d), jnp.float32),
  )(xp, W, degp)

  agg_kernel = pl.kernel(
      functools.partial(_agg_body, nk, rpt),
      out_type=jax.ShapeDtypeStruct((NC, np_, d), jnp.float32),
      mesh=mesh,
      scratch_types=[
          pltpu.VMEM_SHARED((np_, d), jnp.float32),
          pltpu.VMEM((NI, 2, CH), jnp.int32),
          pltpu.VMEM((NB, CH, d), jnp.float32),
          pltpu.SemaphoreType.DMA((NI,)),
          pltpu.SemaphoreType.DMA((NB,)),
      ],
  )
  part = agg_kernel(sd, hs)

  blk2 = 2000
  out = pl.pallas_call(
      _out_body,
      grid=(n // blk2,),
      in_specs=[
          pl.BlockSpec((NC, blk2, d), lambda i: (0, i, 0)),
          pl.BlockSpec((blk2, d), lambda i: (i, 0)),
          pl.BlockSpec((blk2, NC), lambda i: (i, 0)),
          pl.BlockSpec((1, d), lambda i: (0, 0)),
      ],
      out_specs=pl.BlockSpec((blk2, d), lambda i: (i, 0)),
      out_shape=jax.ShapeDtypeStruct((n, d), jnp.float32),
  )(part, hs, degp, b.reshape(1, d))

  return out


# speedup vs baseline: 30.5060x; 1.7331x over previous
import functools

import jax
import jax.numpy as jnp
from jax import lax
from jax.experimental import pallas as pl
from jax.experimental.pallas import tpu as pltpu
from jax.experimental.pallas import tpu_sc as plsc

NC = 2
NS = 16
NW = NC * NS
CH = 128
CHA = 64


def _fill_1d(ref, n, value):
  v = jnp.full((16,), value, dtype=jnp.float32)

  def body(i, _):
    ref[pl.ds(i * 16, 16)] = v
    return 0

  lax.fori_loop(0, n // 16, body, 0)


def _fill_2d(ref, rows, cols, value):
  v = jnp.full((16,), value, dtype=jnp.float32)

  def body(i, _):
    for j in range(cols // 16):
      ref[i, pl.ds(j * 16, 16)] = v
    return 0

  lax.fori_loop(0, rows, body, 0)


def _deg_body(nk, rpt, dst_hbm, deg_hbm, deg_acc, didx_v, ones_v, zeros_v,
              sem):
  cid = lax.axis_index("c")
  sid = lax.axis_index("s")
  wid = cid * NS + sid

  _fill_1d(ones_v, CH, 1.0)
  _fill_1d(zeros_v, rpt, 0.0)
  pltpu.sync_copy(zeros_v, deg_acc.at[pl.ds(sid * rpt, rpt)])
  pltpu.sync_copy(dst_hbm.at[wid], didx_v)
  plsc.subcore_barrier()

  def fire(k, _):
    pltpu.async_copy(ones_v, deg_acc.at[didx_v.at[k]], sem, add=True)
    return 0

  lax.fori_loop(0, nk, fire, 0)

  def drain(k, _):
    pltpu.make_async_copy(ones_v, deg_acc.at[didx_v.at[0]], sem).wait()
    return 0

  lax.fori_loop(0, nk, drain, 0)
  plsc.subcore_barrier()
  pltpu.sync_copy(deg_acc.at[pl.ds(sid * rpt, rpt)],
                  deg_hbm.at[cid, pl.ds(sid * rpt, rpt)])


NB = 4
NI = 8
GAP = 2


def _agg_body(nk, rpt, sd_hbm, hs_hbm, part_hbm,
              acc, idxb_v, rows_v, isem, gsem, ssem):
  cid = lax.axis_index("c")
  sid = lax.axis_index("s")
  wid = cid * NS + sid

  _fill_2d(rows_v.at[0], CHA, 128, 0.0)
  nz = rpt // CHA
  for r in range(nz):
    pltpu.sync_copy(rows_v.at[0], acc.at[pl.ds(sid * rpt + r * CHA, CHA)])
  if rpt % CHA:
    pltpu.sync_copy(rows_v.at[0, pl.ds(0, rpt % CHA)],
                    acc.at[pl.ds(sid * rpt + nz * CHA, rpt % CHA)])
  plsc.subcore_barrier()

  for k in range(NI - GAP):
    pltpu.async_copy(sd_hbm.at[wid, k], idxb_v.at[k], isem.at[k])
  for k in range(GAP):
    pltpu.make_async_copy(
        sd_hbm.at[wid, k], idxb_v.at[k], isem.at[k]).wait()
    pltpu.async_copy(hs_hbm.at[idxb_v.at[k, 0]], rows_v.at[k], gsem.at[k])

  def step(k, _):
    b = lax.rem(k, NB)
    i = lax.rem(k, NI)
    pltpu.make_async_copy(
        hs_hbm.at[idxb_v.at[i, 0]], rows_v.at[b], gsem.at[b]).wait()
    pltpu.async_copy(rows_v.at[b], acc.at[idxb_v.at[i, 1]], ssem.at[b],
                     add=True)

    @pl.when(k >= GAP)
    def _():
      bo = lax.rem(k + GAP, NB)
      pltpu.make_async_copy(
          rows_v.at[bo], acc.at[idxb_v.at[0, 1]], ssem.at[bo]).wait()

    @pl.when(k + NI - GAP < nk)
    def _():
      ki = k + NI - GAP
      ii = lax.rem(ki, NI)
      pltpu.async_copy(sd_hbm.at[wid, ki], idxb_v.at[ii], isem.at[ii])

    @pl.when(k + GAP < nk)
    def _():
      kg = k + GAP
      ig = lax.rem(kg, NI)
      bg = lax.rem(kg, NB)
      pltpu.make_async_copy(
          sd_hbm.at[wid, ig], idxb_v.at[ig], isem.at[ig]).wait()
      pltpu.async_copy(hs_hbm.at[idxb_v.at[ig, 0]], rows_v.at[bg],
                       gsem.at[bg])

    return 0

  lax.fori_loop(0, nk, step, 0)
  for t in range(GAP):
    bo = (nk - GAP + t) % NB
    pltpu.make_async_copy(
        rows_v.at[bo], acc.at[idxb_v.at[0, 1]], ssem.at[bo]).wait()
  plsc.subcore_barrier()
  pltpu.sync_copy(acc.at[pl.ds(sid * rpt, rpt)],
                  part_hbm.at[cid, pl.ds(sid * rpt, rpt)])


def _hs_body(x_ref, w_ref, degp_ref, hs_ref):
  deg = degp_ref[:, 0] + degp_ref[:, 1] + 1.0
  dinv = lax.rsqrt(deg)
  h = jnp.dot(x_ref[...], w_ref[...], preferred_element_type=jnp.float32)
  hs_ref[...] = h * dinv[:, None]


def _out_body(part_ref, hs_ref, degp_ref, b_ref, out_ref):
  deg = degp_ref[:, 0] + degp_ref[:, 1] + 1.0
  dinv = lax.rsqrt(deg)
  s = part_ref[0] + part_ref[1] + hs_ref[...]
  out_ref[...] = s * dinv[:, None] + b_ref[0, :]


def kernel(x, edge_index, W, b):
  n, d = x.shape
  e = edge_index.shape[1]

  rpt = -(-(n + 1) // NS)
  rpt = -(-rpt // CH) * CH
  np_ = NS * rpt
  rpa = 8 * (-(-(n + 1) // (8 * NS)))
  npa = NS * rpa

  nkd = -(-e // (NW * CH))
  epd = NW * nkd * CH
  dstd = jnp.concatenate(
      [edge_index[1], jnp.full((epd - e,), n, dtype=jnp.int32)]
  ).reshape(NW, nkd, CH)

  nk = -(-e // (NW * CHA))
  ep = NW * nk * CHA
  src = jnp.concatenate(
      [edge_index[0], jnp.full((ep - e,), n, dtype=jnp.int32)]
  ).reshape(NW, nk, CHA)
  dst = jnp.concatenate(
      [edge_index[1], jnp.full((ep - e,), n, dtype=jnp.int32)]
  ).reshape(NW, nk, CHA)
  sd = jnp.stack([src, dst], axis=2)
  xp = jnp.concatenate(
      [x, jnp.zeros((np_ - n, d), dtype=x.dtype)])

  mesh = plsc.VectorSubcoreMesh(core_axis_name="c", subcore_axis_name="s")

  deg_kernel = pl.kernel(
      functools.partial(_deg_body, nkd, rpt),
      out_type=jax.ShapeDtypeStruct((NC, np_), jnp.float32),
      mesh=mesh,
      scratch_types=[
          pltpu.VMEM_SHARED((np_,), jnp.float32),
          pltpu.VMEM((nkd, CH), jnp.int32),
          pltpu.VMEM((CH,), jnp.float32),
          pltpu.VMEM((rpt,), jnp.float32),
          pltpu.SemaphoreType.DMA,
      ],
  )
  degp = deg_kernel(dstd).T

  blk1 = 512
  hs = pl.pallas_call(
      _hs_body,
      grid=(np_ // blk1,),
      in_specs=[
          pl.BlockSpec((blk1, d), lambda i: (i, 0)),
          pl.BlockSpec((d, d), lambda i: (0, 0)),
          pl.BlockSpec((blk1, NC), lambda i: (i, 0)),
      ],
      out_specs=pl.BlockSpec((blk1, d), lambda i: (i, 0)),
      out_shape=jax.ShapeDtypeStruct((np_, d), jnp.float32),
  )(xp, W, degp)

  agg_kernel = pl.kernel(
      functools.partial(_agg_body, nk, rpa),
      out_type=jax.ShapeDtypeStruct((NC, npa, d), jnp.float32),
      mesh=mesh,
      scratch_types=[
          pltpu.VMEM_SHARED((npa, d), jnp.float32),
          pltpu.VMEM((NI, 2, CHA), jnp.int32),
          pltpu.VMEM((NB, CHA, d), jnp.float32),
          pltpu.SemaphoreType.DMA((NI,)),
          pltpu.SemaphoreType.DMA((NB,)),
          pltpu.SemaphoreType.DMA((NB,)),
      ],
  )
  part = agg_kernel(sd, hs)

  blk2 = 2000
  out = pl.pallas_call(
      _out_body,
      grid=(n // blk2,),
      in_specs=[
          pl.BlockSpec((NC, blk2, d), lambda i: (0, i, 0)),
          pl.BlockSpec((blk2, d), lambda i: (i, 0)),
          pl.BlockSpec((blk2, NC), lambda i: (i, 0)),
          pl.BlockSpec((1, d), lambda i: (0, 0)),
      ],
      out_specs=pl.BlockSpec((blk2, d), lambda i: (i, 0)),
      out_shape=jax.ShapeDtypeStruct((n, d), jnp.float32),
  )(part, hs, degp, b.reshape(1, d))

  return out
